# Initial kernel scaffold; baseline (speedup 1.0000x reference)
import jax
import jax.numpy as jnp
from jax.experimental import pallas as pl


def kernel(node_rep, edge_rep, edge_attr, edge_index, W1, W2, W3, We, bn_g, bn_b, M1, g1, b1, M2, g2, b2, M3, bias3):
    raise NotImplementedError("write your pallas kernel here")


# speedup vs baseline: 2.2455x; 2.2455x over previous
import functools

import jax
import jax.numpy as jnp
from jax import lax
from jax.experimental import pallas as pl
from jax.experimental.pallas import tpu as pltpu
from jax.experimental.pallas import tpu_sc as plsc

N = 10000
E = 320000
D = 128
DE = 16
H = 256
EPS = 1e-5

NC = 2
NS = 16
NW = NC * NS
EPW = E // NW
BLK = 80
NBLK = EPW // BLK
RPS = 624

_HI = lax.Precision.HIGHEST
_F = jnp.float32


def _dot(a, b):
    return jnp.dot(a, b, preferred_element_type=_F, precision=_HI)


_BN_ROWS = 2000

def _node_mm_body(x_ref, w1_ref, w2_ref, o1_ref, o2_ref):
    x = x_ref[...]
    o1_ref[...] = _dot(x, w1_ref[...])
    o2_ref[...] = _dot(x, w2_ref[...])


def _node_mm(x, w1, w2):
    return pl.pallas_call(
        _node_mm_body,
        grid=(N // _BN_ROWS,),
        in_specs=[
            pl.BlockSpec((_BN_ROWS, D), lambda i: (i, 0)),
            pl.BlockSpec((D, D), lambda i: (0, 0)),
            pl.BlockSpec((D, D), lambda i: (0, 0)),
        ],
        out_specs=(pl.BlockSpec((_BN_ROWS, D), lambda i: (i, 0)),
                   pl.BlockSpec((_BN_ROWS, D), lambda i: (i, 0))),
        out_shape=(jax.ShapeDtypeStruct((N, D), _F),
                   jax.ShapeDtypeStruct((N, D), _F)),
    )(x, w1, w2)


_BE = 4000

def _edge_mm_body(er_ref, ea_ref, w3_ref, we_ref, o_ref):
    o_ref[...] = _dot(er_ref[...], w3_ref[...]) + _dot(ea_ref[...], we_ref[...])


def _edge_mm(edge_rep, edge_attr, w3, we):
    return pl.pallas_call(
        _edge_mm_body,
        grid=(E // _BE,),
        in_specs=[
            pl.BlockSpec((_BE, D), lambda i: (i, 0)),
            pl.BlockSpec((_BE, DE), lambda i: (i, 0)),
            pl.BlockSpec((D, D), lambda i: (0, 0)),
            pl.BlockSpec((DE, D), lambda i: (0, 0)),
        ],
        out_specs=pl.BlockSpec((_BE, D), lambda i: (i, 0)),
        out_shape=jax.ShapeDtypeStruct((E, D), _F),
    )(edge_rep, edge_attr, w3, we)


_mesh = plsc.VectorSubcoreMesh(core_axis_name="c", subcore_axis_name="s")


@functools.partial(
    pl.kernel,
    mesh=_mesh,
    out_type=(jax.ShapeDtypeStruct((E, D), _F),
              jax.ShapeDtypeStruct((NW, 2, D), _F)),
    scratch_types=[
        pltpu.VMEM((BLK,), jnp.int32),
        pltpu.VMEM((BLK,), jnp.int32),
        pltpu.VMEM((BLK, D), _F),
        pltpu.VMEM((BLK, D), _F),
        pltpu.VMEM((BLK, D), _F),
        pltpu.VMEM((2, D), _F),
        pltpu.SemaphoreType.DMA,
        pltpu.SemaphoreType.DMA,
        pltpu.SemaphoreType.DMA,
    ],
)
def _sc_pass1(xw1, xw2, edense, src_hbm, dst_hbm, msg_out, acc_out,
              src_v, dst_v, g1_v, g2_v, ed_v, st_v, sem1, sem2, sem3):
    c = lax.axis_index("c")
    s = lax.axis_index("s")
    wid = s * NC + c
    base = wid * EPW

    def block(i, carry):
        off = base + i * BLK
        pltpu.sync_copy(src_hbm.at[pl.ds(off, BLK)], src_v)
        pltpu.sync_copy(dst_hbm.at[pl.ds(off, BLK)], dst_v)
        cp1 = pltpu.async_copy(xw1.at[src_v], g1_v, sem1)
        cp2 = pltpu.async_copy(xw2.at[dst_v], g2_v, sem2)
        cp3 = pltpu.async_copy(edense.at[pl.ds(off, BLK)], ed_v, sem3)
        cp1.wait()
        cp2.wait()
        cp3.wait()

        def row(r, acc):
            new = list(acc)
            for f in range(8):
                sl = pl.ds(f * 16, 16)
                v = g1_v[r, sl] + g2_v[r, sl] + ed_v[r, sl]
                g1_v[r, sl] = v
                new[f] = acc[f] + v
                new[8 + f] = acc[8 + f] + v * v
            return tuple(new)

        carry = lax.fori_loop(0, BLK, row, carry)
        pltpu.sync_copy(g1_v, msg_out.at[pl.ds(off, BLK)])
        return carry

    zero = jnp.zeros((16,), _F)
    acc = lax.fori_loop(0, NBLK, block, tuple(zero for _ in range(16)))
    for f in range(8):
        st_v[0, pl.ds(f * 16, 16)] = acc[f]
        st_v[1, pl.ds(f * 16, 16)] = acc[8 + f]
    pltpu.sync_copy(st_v, acc_out.at[wid])


@functools.partial(
    pl.kernel,
    mesh=_mesh,
    out_type=jax.ShapeDtypeStruct((NC, N, D), _F),
    scratch_types=[
        pltpu.VMEM((BLK,), jnp.int32),
        pltpu.VMEM((BLK, D), _F),
        pltpu.VMEM((2, D), _F),
        pltpu.VMEM((104, D), _F),
        pltpu.VMEM_SHARED((N, D), _F),
        pltpu.SemaphoreType.DMA,
    ],
)
def _sc_pass2(msg_hbm, dst_hbm, ab_hbm, y_out,
              idx_v, m_v, ab_v, z_v, ysh, sem):
    c = lax.axis_index("c")
    s = lax.axis_index("s")
    wid = s * NC + c
    base = wid * EPW

    pltpu.sync_copy(ab_hbm, ab_v)
    a = [ab_v[0, pl.ds(f * 16, 16)] for f in range(8)]
    b = [ab_v[1, pl.ds(f * 16, 16)] for f in range(8)]

    zero = jnp.zeros((16,), _F)

    def zrow(r, _):
        for f in range(8):
            z_v[r, pl.ds(f * 16, 16)] = zero
        return 0

    lax.fori_loop(0, 104, zrow, 0)

    def zchunk(k, _):
        pltpu.sync_copy(z_v, ysh.at[pl.ds(s * RPS + k * 104, 104)])
        return 0

    lax.fori_loop(0, RPS // 104, zchunk, 0)

    @pl.when(s == NS - 1)
    def _zero_tail():
        pltpu.sync_copy(z_v.at[pl.ds(0, 16)], ysh.at[pl.ds(NS * RPS, 16)])

    plsc.subcore_barrier()

    def block(i, _):
        off = base + i * BLK
        pltpu.sync_copy(dst_hbm.at[pl.ds(off, BLK)], idx_v)
        pltpu.sync_copy(msg_hbm.at[pl.ds(off, BLK)], m_v)

        def row(r, _2):
            for f in range(8):
                sl = pl.ds(f * 16, 16)
                m_v[r, sl] = jnp.maximum(m_v[r, sl] * a[f] + b[f], 0.0)
            return 0

        lax.fori_loop(0, BLK, row, 0)
        pltpu.sync_copy(m_v, ysh.at[idx_v], add=True)
        return 0

    lax.fori_loop(0, NBLK, block, 0)
    plsc.subcore_barrier()
    pltpu.sync_copy(ysh.at[pl.ds(s * RPS, RPS)],
                    y_out.at[c, pl.ds(s * RPS, RPS)])

    @pl.when(s == NS - 1)
    def _out_tail():
        pltpu.sync_copy(ysh.at[pl.ds(NS * RPS, 16)],
                        y_out.at[c, pl.ds(NS * RPS, 16)])


_BR = 2000
_MG = N // _BR


def _stats(t):
    s0 = jnp.sum(t, axis=0, keepdims=True)
    s1 = jnp.sum(t * t, axis=0, keepdims=True)
    return jnp.concatenate([s0, s1], axis=0)


def _mlp_a_body(y2_ref, m1_ref, t1_ref, st_ref):
    y = y2_ref[0] + y2_ref[1]
    t = _dot(y, m1_ref[...])
    t1_ref[...] = t
    st_ref[0] = _stats(t)


def _mlp_a(y2, m1):
    return pl.pallas_call(
        _mlp_a_body,
        grid=(_MG,),
        in_specs=[
            pl.BlockSpec((NC, _BR, D), lambda i: (0, i, 0)),
            pl.BlockSpec((D, H), lambda i: (0, 0)),
        ],
        out_specs=(pl.BlockSpec((_BR, H), lambda i: (i, 0)),
                   pl.BlockSpec((1, 2, H), lambda i: (i, 0, 0))),
        out_shape=(jax.ShapeDtypeStruct((N, H), _F),
                   jax.ShapeDtypeStruct((_MG, 2, H), _F)),
    )(y2, m1)


def _mlp_b_body(t1_ref, a_ref, b_ref, m2_ref, t2_ref, st_ref):
    h = jnp.maximum(t1_ref[...] * a_ref[...] + b_ref[...], 0.0)
    t = _dot(h, m2_ref[...])
    t2_ref[...] = t
    st_ref[0] = _stats(t)


def _mlp_b(t1, a, b, m2):
    return pl.pallas_call(
        _mlp_b_body,
        grid=(_MG,),
        in_specs=[
            pl.BlockSpec((_BR, H), lambda i: (i, 0)),
            pl.BlockSpec((H,), lambda i: (0,)),
            pl.BlockSpec((H,), lambda i: (0,)),
            pl.BlockSpec((H, H), lambda i: (0, 0)),
        ],
        out_specs=(pl.BlockSpec((_BR, H), lambda i: (i, 0)),
                   pl.BlockSpec((1, 2, H), lambda i: (i, 0, 0))),
        out_shape=(jax.ShapeDtypeStruct((N, H), _F),
                   jax.ShapeDtypeStruct((_MG, 2, H), _F)),
    )(t1, a, b, m2)


def _mlp_c_body(t2_ref, a_ref, b_ref, m3_ref, bias_ref, o_ref):
    h = jnp.maximum(t2_ref[...] * a_ref[...] + b_ref[...], 0.0)
    o_ref[...] = _dot(h, m3_ref[...]) + bias_ref[...]


def _mlp_c(t2, a, b, m3, bias3):
    return pl.pallas_call(
        _mlp_c_body,
        grid=(_MG,),
        in_specs=[
            pl.BlockSpec((_BR, H), lambda i: (i, 0)),
            pl.BlockSpec((H,), lambda i: (0,)),
            pl.BlockSpec((H,), lambda i: (0,)),
            pl.BlockSpec((H, D), lambda i: (0, 0)),
            pl.BlockSpec((D,), lambda i: (0,)),
        ],
        out_specs=pl.BlockSpec((_BR, D), lambda i: (i, 0)),
        out_shape=jax.ShapeDtypeStruct((N, D), _F),
    )(t2, a, b, m3, bias3)


def kernel(node_rep, edge_rep, edge_attr, edge_index, W1, W2, W3, We,
           bn_g, bn_b, M1, g1, b1, M2, g2, b2, M3, bias3):
    src = edge_index[0].astype(jnp.int32)
    dst = edge_index[1].astype(jnp.int32)

    xw1, xw2 = _node_mm(node_rep, W1, W2)
    edense = _edge_mm(edge_rep, edge_attr, W3, We)

    msgs, acc = _sc_pass1(xw1, xw2, edense, src, dst)

    scale, shift = _bn_ab(acc, bn_g, bn_b, E)
    ab = jnp.stack([scale, shift])

    y2 = _sc_pass2(msgs, dst, ab)

    t1, st1 = _mlp_a(y2, M1)
    a1, s1 = _bn_ab(st1, g1, b1, N)
    t2, st2 = _mlp_b(t1, a1, s1, M2)
    a2, s2 = _bn_ab(st2, g2, b2, N)
    return _mlp_c(t2, a2, s2, M3, bias3)


def _bn_ab(st, g, b, n):
    ssum = jnp.sum(st[:, 0, :], axis=0)
    ssq = jnp.sum(st[:, 1, :], axis=0)
    mean = ssum / n
    var = ssq / n - mean * mean
    scale = g * lax.rsqrt(var + EPS)
    return scale, b - mean * scale


# speedup vs baseline: 3.4973x; 1.5575x over previous
import functools

import jax
import jax.numpy as jnp
from jax import lax
from jax.experimental import pallas as pl
from jax.experimental.pallas import tpu as pltpu
from jax.experimental.pallas import tpu_sc as plsc

N = 10000
E = 320000
D = 128
DE = 16
H = 256
EPS = 1e-5

NC = 2
NS = 16
NW = NC * NS
EPW = E // NW
BLK = 80
NBLK = EPW // BLK
RPS = 624

_HI = lax.Precision.HIGHEST
_F = jnp.float32


def _dot(a, b):
    return jnp.dot(a, b, preferred_element_type=_F, precision=_HI)


_BN_ROWS = 2000

def _node_mm_body(x_ref, w1_ref, w2_ref, o1_ref, o2_ref):
    x = x_ref[...]
    o1_ref[...] = _dot(x, w1_ref[...])
    o2_ref[...] = _dot(x, w2_ref[...])


def _node_mm(x, w1, w2):
    return pl.pallas_call(
        _node_mm_body,
        grid=(N // _BN_ROWS,),
        in_specs=[
            pl.BlockSpec((_BN_ROWS, D), lambda i: (i, 0)),
            pl.BlockSpec((D, D), lambda i: (0, 0)),
            pl.BlockSpec((D, D), lambda i: (0, 0)),
        ],
        out_specs=(pl.BlockSpec((_BN_ROWS, D), lambda i: (i, 0)),
                   pl.BlockSpec((_BN_ROWS, D), lambda i: (i, 0))),
        out_shape=(jax.ShapeDtypeStruct((N, D), _F),
                   jax.ShapeDtypeStruct((N, D), _F)),
    )(x, w1, w2)


_BE = 4000

def _edge_mm_body(er_ref, ea_ref, w3_ref, we_ref, o_ref):
    o_ref[...] = _dot(er_ref[...], w3_ref[...]) + _dot(ea_ref[...], we_ref[...])


def _edge_mm(edge_rep, edge_attr, w3, we):
    return pl.pallas_call(
        _edge_mm_body,
        grid=(E // _BE,),
        in_specs=[
            pl.BlockSpec((_BE, D), lambda i: (i, 0)),
            pl.BlockSpec((_BE, DE), lambda i: (i, 0)),
            pl.BlockSpec((D, D), lambda i: (0, 0)),
            pl.BlockSpec((DE, D), lambda i: (0, 0)),
        ],
        out_specs=pl.BlockSpec((_BE, D), lambda i: (i, 0)),
        out_shape=jax.ShapeDtypeStruct((E, D), _F),
    )(edge_rep, edge_attr, w3, we)


_mesh = plsc.VectorSubcoreMesh(core_axis_name="c", subcore_axis_name="s")


@functools.partial(
    pl.kernel,
    mesh=_mesh,
    out_type=(jax.ShapeDtypeStruct((E, D), _F),
              jax.ShapeDtypeStruct((NW, 2, D), _F)),
    scratch_types=[
        (pltpu.VMEM((BLK,), jnp.int32),) * 2,
        (pltpu.VMEM((BLK,), jnp.int32),) * 2,
        (pltpu.VMEM((BLK, D), _F),) * 2,
        (pltpu.VMEM((BLK, D), _F),) * 2,
        (pltpu.VMEM((BLK, D), _F),) * 2,
        pltpu.VMEM((2, D), _F),
        (pltpu.SemaphoreType.DMA,) * 2,
        (pltpu.SemaphoreType.DMA,) * 2,
        (pltpu.SemaphoreType.DMA,) * 2,
    ],
)
def _sc_pass1(xw1, xw2, edense, src_hbm, dst_hbm, msg_out, acc_out,
              src_v, dst_v, g1_v, g2_v, ed_v, st_v, sem_i, sem_g, sem_o):
    c = lax.axis_index("c")
    s = lax.axis_index("s")
    wid = s * NC + c
    base = wid * EPW

    def idx_copies(j, b):
        off = base + j * BLK
        return (pltpu.make_async_copy(src_hbm.at[pl.ds(off, BLK)], src_v[b], sem_i[b]),
                pltpu.make_async_copy(dst_hbm.at[pl.ds(off, BLK)], dst_v[b], sem_i[b]))

    def gat_copies(j, b):
        off = base + j * BLK
        return (pltpu.make_async_copy(xw1.at[src_v[b]], g1_v[b], sem_g[b]),
                pltpu.make_async_copy(xw2.at[dst_v[b]], g2_v[b], sem_g[b]),
                pltpu.make_async_copy(edense.at[pl.ds(off, BLK)], ed_v[b], sem_g[b]))

    def out_copy(j, b):
        off = base + j * BLK
        return pltpu.make_async_copy(g1_v[b], msg_out.at[pl.ds(off, BLK)], sem_o[b])

    for cp in idx_copies(0, 0) + idx_copies(1, 1):
        cp.start()
    for cp in idx_copies(0, 0):
        cp.wait()
    for cp in gat_copies(0, 0):
        cp.start()

    def step(j, b, carry):
        q = 1 - b
        for cp in gat_copies(j, b):
            cp.wait()

        @pl.when(j + 1 < NBLK)
        def _launch_next():
            for cp in idx_copies(j + 1, q):
                cp.wait()

            @pl.when(j >= 1)
            def _drain_prev_out():
                out_copy(j - 1, q).wait()

            for cp in gat_copies(j + 1, q):
                cp.start()

        @pl.when(j + 2 < NBLK)
        def _prefetch_idx():
            for cp in idx_copies(j + 2, b):
                cp.start()

        def row(r, acc):
            new = list(acc)
            for f in range(8):
                sl = pl.ds(f * 16, 16)
                v = g1_v[b][r, sl] + g2_v[b][r, sl] + ed_v[b][r, sl]
                g1_v[b][r, sl] = v
                new[f] = acc[f] + v
                new[8 + f] = acc[8 + f] + v * v
            return tuple(new)

        carry = lax.fori_loop(0, BLK, row, carry)
        out_copy(j, b).start()
        return carry

    def pair(io, carry):
        carry = step(2 * io, 0, carry)
        return step(2 * io + 1, 1, carry)

    zero = jnp.zeros((16,), _F)
    acc = lax.fori_loop(0, NBLK // 2, pair, tuple(zero for _ in range(16)))
    acc = step(NBLK - 1, 0, acc)
    out_copy(NBLK - 2, 1).wait()
    out_copy(NBLK - 1, 0).wait()
    for f in range(8):
        st_v[0, pl.ds(f * 16, 16)] = acc[f]
        st_v[1, pl.ds(f * 16, 16)] = acc[8 + f]
    pltpu.sync_copy(st_v, acc_out.at[wid])


@functools.partial(
    pl.kernel,
    mesh=_mesh,
    out_type=jax.ShapeDtypeStruct((NC, N, D), _F),
    scratch_types=[
        (pltpu.VMEM((BLK,), jnp.int32),) * 3,
        (pltpu.VMEM((BLK, D), _F),) * 3,
        pltpu.VMEM((2, D), _F),
        pltpu.VMEM((104, D), _F),
        pltpu.VMEM_SHARED((N, D), _F),
        (pltpu.SemaphoreType.DMA,) * 3,
        (pltpu.SemaphoreType.DMA,) * 3,
    ],
)
def _sc_pass2(msg_hbm, dst_hbm, ab_hbm, y_out,
              idx_v, m_v, ab_v, z_v, ysh, sem_i, sem_s):
    c = lax.axis_index("c")
    s = lax.axis_index("s")
    wid = s * NC + c
    base = wid * EPW

    pltpu.sync_copy(ab_hbm, ab_v)
    a = [ab_v[0, pl.ds(f * 16, 16)] for f in range(8)]
    b = [ab_v[1, pl.ds(f * 16, 16)] for f in range(8)]

    zero = jnp.zeros((16,), _F)

    def zrow(r, _):
        for f in range(8):
            z_v[r, pl.ds(f * 16, 16)] = zero
        return 0

    lax.fori_loop(0, 104, zrow, 0)

    def zchunk(k, _):
        pltpu.sync_copy(z_v, ysh.at[pl.ds(s * RPS + k * 104, 104)])
        return 0

    lax.fori_loop(0, RPS // 104, zchunk, 0)

    @pl.when(s == NS - 1)
    def _zero_tail():
        pltpu.sync_copy(z_v.at[pl.ds(0, 16)], ysh.at[pl.ds(NS * RPS, 16)])

    plsc.subcore_barrier()

    def in_copies(j, p):
        off = base + j * BLK
        return (pltpu.make_async_copy(dst_hbm.at[pl.ds(off, BLK)], idx_v[p], sem_i[p]),
                pltpu.make_async_copy(msg_hbm.at[pl.ds(off, BLK)], m_v[p], sem_i[p]))

    def sc_copy(p):
        return pltpu.async_copy(m_v[p], ysh.at[idx_v[p]], sem_s[p], add=True)

    def sc_wait(p):
        pltpu.make_async_copy(m_v[p], ysh.at[idx_v[p]], sem_s[p]).wait()

    for cp in in_copies(0, 0) + in_copies(1, 1):
        cp.start()

    def step(j, p):
        for cp in in_copies(j, p):
            cp.wait()

        def row(r, _2):
            for f in range(8):
                sl = pl.ds(f * 16, 16)
                m_v[p][r, sl] = jnp.maximum(m_v[p][r, sl] * a[f] + b[f], 0.0)
            return 0

        lax.fori_loop(0, BLK, row, 0)

        @pl.when(j >= 1)
        def _drain_prev_scatter():
            sc_wait((p + 2) % 3)

        @pl.when(j + 2 < NBLK)
        def _prefetch_in():
            for cp in in_copies(j + 2, (p + 2) % 3):
                cp.start()

        sc_copy(p)

    def triple(io, _):
        step(3 * io, 0)
        step(3 * io + 1, 1)
        step(3 * io + 2, 2)
        return 0

    lax.fori_loop(0, NBLK // 3, triple, 0)
    step(NBLK - 2, 0)
    step(NBLK - 1, 1)
    sc_wait(1)
    plsc.subcore_barrier()
    pltpu.sync_copy(ysh.at[pl.ds(s * RPS, RPS)],
                    y_out.at[c, pl.ds(s * RPS, RPS)])

    @pl.when(s == NS - 1)
    def _out_tail():
        pltpu.sync_copy(ysh.at[pl.ds(NS * RPS, 16)],
                        y_out.at[c, pl.ds(NS * RPS, 16)])


_BR = 2000
_MG = N // _BR


def _stats(t):
    s0 = jnp.sum(t, axis=0, keepdims=True)
    s1 = jnp.sum(t * t, axis=0, keepdims=True)
    return jnp.concatenate([s0, s1], axis=0)


def _mlp_a_body(y2_ref, m1_ref, t1_ref, st_ref):
    y = y2_ref[0] + y2_ref[1]
    t = _dot(y, m1_ref[...])
    t1_ref[...] = t
    st_ref[0] = _stats(t)


def _mlp_a(y2, m1):
    return pl.pallas_call(
        _mlp_a_body,
        grid=(_MG,),
        in_specs=[
            pl.BlockSpec((NC, _BR, D), lambda i: (0, i, 0)),
            pl.BlockSpec((D, H), lambda i: (0, 0)),
        ],
        out_specs=(pl.BlockSpec((_BR, H), lambda i: (i, 0)),
                   pl.BlockSpec((1, 2, H), lambda i: (i, 0, 0))),
        out_shape=(jax.ShapeDtypeStruct((N, H), _F),
                   jax.ShapeDtypeStruct((_MG, 2, H), _F)),
    )(y2, m1)


def _mlp_b_body(t1_ref, a_ref, b_ref, m2_ref, t2_ref, st_ref):
    h = jnp.maximum(t1_ref[...] * a_ref[...] + b_ref[...], 0.0)
    t = _dot(h, m2_ref[...])
    t2_ref[...] = t
    st_ref[0] = _stats(t)


def _mlp_b(t1, a, b, m2):
    return pl.pallas_call(
        _mlp_b_body,
        grid=(_MG,),
        in_specs=[
            pl.BlockSpec((_BR, H), lambda i: (i, 0)),
            pl.BlockSpec((H,), lambda i: (0,)),
            pl.BlockSpec((H,), lambda i: (0,)),
            pl.BlockSpec((H, H), lambda i: (0, 0)),
        ],
        out_specs=(pl.BlockSpec((_BR, H), lambda i: (i, 0)),
                   pl.BlockSpec((1, 2, H), lambda i: (i, 0, 0))),
        out_shape=(jax.ShapeDtypeStruct((N, H), _F),
                   jax.ShapeDtypeStruct((_MG, 2, H), _F)),
    )(t1, a, b, m2)


def _mlp_c_body(t2_ref, a_ref, b_ref, m3_ref, bias_ref, o_ref):
    h = jnp.maximum(t2_ref[...] * a_ref[...] + b_ref[...], 0.0)
    o_ref[...] = _dot(h, m3_ref[...]) + bias_ref[...]


def _mlp_c(t2, a, b, m3, bias3):
    return pl.pallas_call(
        _mlp_c_body,
        grid=(_MG,),
        in_specs=[
            pl.BlockSpec((_BR, H), lambda i: (i, 0)),
            pl.BlockSpec((H,), lambda i: (0,)),
            pl.BlockSpec((H,), lambda i: (0,)),
            pl.BlockSpec((H, D), lambda i: (0, 0)),
            pl.BlockSpec((D,), lambda i: (0,)),
        ],
        out_specs=pl.BlockSpec((_BR, D), lambda i: (i, 0)),
        out_shape=jax.ShapeDtypeStruct((N, D), _F),
    )(t2, a, b, m3, bias3)


def kernel(node_rep, edge_rep, edge_attr, edge_index, W1, W2, W3, We,
           bn_g, bn_b, M1, g1, b1, M2, g2, b2, M3, bias3):
    src = edge_index[0].astype(jnp.int32)
    dst = edge_index[1].astype(jnp.int32)

    xw1, xw2 = _node_mm(node_rep, W1, W2)
    edense = _edge_mm(edge_rep, edge_attr, W3, We)

    msgs, acc = _sc_pass1(xw1, xw2, edense, src, dst)

    scale, shift = _bn_ab(acc, bn_g, bn_b, E)
    ab = jnp.stack([scale, shift])

    y2 = _sc_pass2(msgs, dst, ab)

    t1, st1 = _mlp_a(y2, M1)
    a1, s1 = _bn_ab(st1, g1, b1, N)
    t2, st2 = _mlp_b(t1, a1, s1, M2)
    a2, s2 = _bn_ab(st2, g2, b2, N)
    return _mlp_c(t2, a2, s2, M3, bias3)


def _bn_ab(st, g, b, n):
    ssum = jnp.sum(st[:, 0, :], axis=0)
    ssq = jnp.sum(st[:, 1, :], axis=0)
    mean = ssum / n
    var = ssq / n - mean * mean
    scale = g * lax.rsqrt(var + EPS)
    return scale, b - mean * scale


# speedup vs baseline: 3.6800x; 1.0522x over previous
import functools

import jax
import jax.numpy as jnp
from jax import lax
from jax.experimental import pallas as pl
from jax.experimental.pallas import tpu as pltpu
from jax.experimental.pallas import tpu_sc as plsc

N = 10000
E = 320000
D = 128
DE = 16
H = 256
EPS = 1e-5

NC = 2
NS = 16
NW = NC * NS
EPW = E // NW
BLK = 80
NBLK = EPW // BLK
RPS = 624
BLK2 = 40
NBLK2 = EPW // BLK2

_HI = lax.Precision.HIGHEST
_F = jnp.float32


def _dot(a, b):
    return jnp.dot(a, b, preferred_element_type=_F, precision=_HI)


_BN_ROWS = 2000

def _node_mm_body(x_ref, w1_ref, w2_ref, o1_ref, o2_ref):
    x = x_ref[...]
    o1_ref[...] = _dot(x, w1_ref[...])
    o2_ref[...] = _dot(x, w2_ref[...])


def _node_mm(x, w1, w2):
    return pl.pallas_call(
        _node_mm_body,
        grid=(N // _BN_ROWS,),
        in_specs=[
            pl.BlockSpec((_BN_ROWS, D), lambda i: (i, 0)),
            pl.BlockSpec((D, D), lambda i: (0, 0)),
            pl.BlockSpec((D, D), lambda i: (0, 0)),
        ],
        out_specs=(pl.BlockSpec((_BN_ROWS, D), lambda i: (i, 0)),
                   pl.BlockSpec((_BN_ROWS, D), lambda i: (i, 0))),
        out_shape=(jax.ShapeDtypeStruct((N, D), _F),
                   jax.ShapeDtypeStruct((N, D), _F)),
    )(x, w1, w2)


_BE = 3200

def _edge_mm_body(er_ref, eat_ref, w3_ref, we_ref, o_ref):
    ea_part = lax.dot_general(eat_ref[...], we_ref[...],
                              (((0,), (0,)), ((), ())),
                              preferred_element_type=_F, precision=_HI)
    o_ref[...] = _dot(er_ref[...], w3_ref[...]) + ea_part


def _edge_mm(edge_rep, edge_attr_t, w3, we):
    return pl.pallas_call(
        _edge_mm_body,
        grid=(E // _BE,),
        in_specs=[
            pl.BlockSpec((_BE, D), lambda i: (i, 0)),
            pl.BlockSpec((DE, _BE), lambda i: (0, i)),
            pl.BlockSpec((D, D), lambda i: (0, 0)),
            pl.BlockSpec((DE, D), lambda i: (0, 0)),
        ],
        out_specs=pl.BlockSpec((_BE, D), lambda i: (i, 0)),
        out_shape=jax.ShapeDtypeStruct((E, D), _F),
    )(edge_rep, edge_attr_t, w3, we)


_mesh = plsc.VectorSubcoreMesh(core_axis_name="c", subcore_axis_name="s")


def _bf16_round_bits(v):
    i = lax.bitcast_convert_type(v, jnp.int32)
    rnd = jnp.int32(0x7FFF) + ((i >> 16) & jnp.int32(1))
    return ((i + rnd) >> 16) & jnp.int32(0xFFFF)


def _bf16_pack2(v0, v1):
    return _bf16_round_bits(v0) | (_bf16_round_bits(v1) << 16)


def _bf16_unpack2(w):
    f0 = lax.bitcast_convert_type(w << 16, _F)
    f1 = lax.bitcast_convert_type(w & jnp.int32(-65536), _F)
    return f0, f1


@functools.partial(
    pl.kernel,
    mesh=_mesh,
    out_type=(jax.ShapeDtypeStruct((E, D // 2), jnp.int32),
              jax.ShapeDtypeStruct((NW, 2, D), _F)),
    scratch_types=[
        (pltpu.VMEM((BLK,), jnp.int32),) * 2,
        (pltpu.VMEM((BLK,), jnp.int32),) * 2,
        (pltpu.VMEM((BLK, D), _F),) * 2,
        (pltpu.VMEM((BLK, D), _F),) * 2,
        (pltpu.VMEM((BLK, D), _F),) * 2,
        (pltpu.VMEM((BLK, D // 2), jnp.int32),) * 2,
        pltpu.VMEM((2, D), _F),
        (pltpu.SemaphoreType.DMA,) * 2,
        (pltpu.SemaphoreType.DMA,) * 2,
        (pltpu.SemaphoreType.DMA,) * 2,
    ],
)
def _sc_pass1(xw1, xw2, edense, src_hbm, dst_hbm, msg_out, acc_out,
              src_v, dst_v, g1_v, g2_v, ed_v, mb_v, st_v, sem_i, sem_g, sem_o):
    c = lax.axis_index("c")
    s = lax.axis_index("s")
    wid = s * NC + c
    base = wid * EPW

    def idx_copies(j, b):
        off = base + j * BLK
        return (pltpu.make_async_copy(src_hbm.at[pl.ds(off, BLK)], src_v[b], sem_i[b]),
                pltpu.make_async_copy(dst_hbm.at[pl.ds(off, BLK)], dst_v[b], sem_i[b]))

    def gat_copies(j, b):
        off = base + j * BLK
        return (pltpu.make_async_copy(xw1.at[src_v[b]], g1_v[b], sem_g[b]),
                pltpu.make_async_copy(xw2.at[dst_v[b]], g2_v[b], sem_g[b]),
                pltpu.make_async_copy(edense.at[pl.ds(off, BLK)], ed_v[b], sem_g[b]))

    def out_copy(j, b):
        off = base + j * BLK
        return pltpu.make_async_copy(mb_v[b], msg_out.at[pl.ds(off, BLK)], sem_o[b])

    for cp in idx_copies(0, 0) + idx_copies(1, 1):
        cp.start()
    for cp in idx_copies(0, 0):
        cp.wait()
    for cp in gat_copies(0, 0):
        cp.start()

    def step(j, b, carry):
        q = 1 - b
        for cp in gat_copies(j, b):
            cp.wait()

        @pl.when(j + 1 < NBLK)
        def _launch_next():
            for cp in idx_copies(j + 1, q):
                cp.wait()

            @pl.when(j >= 1)
            def _drain_prev_out():
                out_copy(j - 1, q).wait()

            for cp in gat_copies(j + 1, q):
                cp.start()

        @pl.when(j + 2 < NBLK)
        def _prefetch_idx():
            for cp in idx_copies(j + 2, b):
                cp.start()

        def row(r, acc):
            new = list(acc)
            vs = []
            for f in range(8):
                sl = pl.ds(f * 16, 16)
                v = g1_v[b][r, sl] + g2_v[b][r, sl] + ed_v[b][r, sl]
                vs.append(v)
                new[f] = acc[f] + v
                new[8 + f] = acc[8 + f] + v * v
            for f in range(4):
                mb_v[b][r, pl.ds(f * 16, 16)] = _bf16_pack2(vs[2 * f], vs[2 * f + 1])
            return tuple(new)

        carry = lax.fori_loop(0, BLK, row, carry)
        out_copy(j, b).start()
        return carry

    def pair(io, carry):
        carry = step(2 * io, 0, carry)
        return step(2 * io + 1, 1, carry)

    zero = jnp.zeros((16,), _F)
    acc = lax.fori_loop(0, NBLK // 2, pair, tuple(zero for _ in range(16)))
    acc = step(NBLK - 1, 0, acc)
    out_copy(NBLK - 2, 1).wait()
    out_copy(NBLK - 1, 0).wait()
    for f in range(8):
        st_v[0, pl.ds(f * 16, 16)] = acc[f]
        st_v[1, pl.ds(f * 16, 16)] = acc[8 + f]
    pltpu.sync_copy(st_v, acc_out.at[wid])


@functools.partial(
    pl.kernel,
    mesh=_mesh,
    out_type=jax.ShapeDtypeStruct((NC, N, D), _F),
    scratch_types=[
        (pltpu.VMEM((BLK2,), jnp.int32),) * 3,
        (pltpu.VMEM((BLK2, D // 2), jnp.int32),) * 3,
        (pltpu.VMEM((BLK2, D), _F),) * 3,
        pltpu.VMEM((2, D), _F),
        pltpu.VMEM_SHARED((N, D), _F),
        (pltpu.SemaphoreType.DMA,) * 3,
        (pltpu.SemaphoreType.DMA,) * 3,
    ],
)
def _sc_pass2(msg_hbm, dst_hbm, ab_hbm, y_out,
              idx_v, m_v, s_v, ab_v, ysh, sem_i, sem_s):
    c = lax.axis_index("c")
    s = lax.axis_index("s")
    wid = s * NC + c
    base = wid * EPW

    pltpu.sync_copy(ab_hbm, ab_v)
    a = [ab_v[0, pl.ds(f * 16, 16)] for f in range(8)]
    b = [ab_v[1, pl.ds(f * 16, 16)] for f in range(8)]

    zero = jnp.zeros((16,), _F)

    def zrow(r, _):
        for f in range(8):
            s_v[0][r, pl.ds(f * 16, 16)] = zero
        return 0

    lax.fori_loop(0, BLK2, zrow, 0)

    def zchunk(k, _):
        pltpu.sync_copy(s_v[0], ysh.at[pl.ds(s * RPS + k * BLK2, BLK2)])
        return 0

    lax.fori_loop(0, 15, zchunk, 0)
    pltpu.sync_copy(s_v[0].at[pl.ds(0, 24)],
                    ysh.at[pl.ds(s * RPS + 15 * BLK2, 24)])

    @pl.when(s == NS - 1)
    def _zero_tail():
        pltpu.sync_copy(s_v[0].at[pl.ds(0, 16)], ysh.at[pl.ds(NS * RPS, 16)])

    plsc.subcore_barrier()

    def in_copies(j, p):
        off = base + j * BLK2
        return (pltpu.make_async_copy(dst_hbm.at[pl.ds(off, BLK2)], idx_v[p], sem_i[p]),
                pltpu.make_async_copy(msg_hbm.at[pl.ds(off, BLK2)], m_v[p], sem_i[p]))

    def sc_copy(p):
        return pltpu.async_copy(s_v[p], ysh.at[idx_v[p]], sem_s[p], add=True)

    def sc_wait(p):
        pltpu.make_async_copy(s_v[p], ysh.at[idx_v[p]], sem_s[p]).wait()

    for cp in in_copies(0, 0) + in_copies(1, 1):
        cp.start()

    def step(j, p):
        for cp in in_copies(j, p):
            cp.wait()

        def row(r, _2):
            for f in range(4):
                u0, u1 = _bf16_unpack2(m_v[p][r, pl.ds(f * 16, 16)])
                s_v[p][r, pl.ds(f * 32, 16)] = jnp.maximum(
                    u0 * a[2 * f] + b[2 * f], 0.0)
                s_v[p][r, pl.ds(f * 32 + 16, 16)] = jnp.maximum(
                    u1 * a[2 * f + 1] + b[2 * f + 1], 0.0)
            return 0

        lax.fori_loop(0, BLK2, row, 0)

        @pl.when(j >= 1)
        def _drain_prev_scatter():
            sc_wait((p + 2) % 3)

        @pl.when(j + 2 < NBLK2)
        def _prefetch_in():
            for cp in in_copies(j + 2, (p + 2) % 3):
                cp.start()

        sc_copy(p)

    def triple(io, _):
        step(3 * io, 0)
        step(3 * io + 1, 1)
        step(3 * io + 2, 2)
        return 0

    lax.fori_loop(0, NBLK2 // 3, triple, 0)
    step(NBLK2 - 1, 0)
    sc_wait(0)
    plsc.subcore_barrier()
    pltpu.sync_copy(ysh.at[pl.ds(s * RPS, RPS)],
                    y_out.at[c, pl.ds(s * RPS, RPS)])

    @pl.when(s == NS - 1)
    def _out_tail():
        pltpu.sync_copy(ysh.at[pl.ds(NS * RPS, 16)],
                        y_out.at[c, pl.ds(NS * RPS, 16)])


_BR = 2000
_MG = N // _BR


def _stats(t):
    s0 = jnp.sum(t, axis=0, keepdims=True)
    s1 = jnp.sum(t * t, axis=0, keepdims=True)
    return jnp.concatenate([s0, s1], axis=0)


def _mlp_a_body(y2_ref, m1_ref, t1_ref, st_ref):
    y = y2_ref[0] + y2_ref[1]
    t = _dot(y, m1_ref[...])
    t1_ref[...] = t
    st_ref[0] = _stats(t)


def _mlp_a(y2, m1):
    return pl.pallas_call(
        _mlp_a_body,
        grid=(_MG,),
        in_specs=[
            pl.BlockSpec((NC, _BR, D), lambda i: (0, i, 0)),
            pl.BlockSpec((D, H), lambda i: (0, 0)),
        ],
        out_specs=(pl.BlockSpec((_BR, H), lambda i: (i, 0)),
                   pl.BlockSpec((1, 2, H), lambda i: (i, 0, 0))),
        out_shape=(jax.ShapeDtypeStruct((N, H), _F),
                   jax.ShapeDtypeStruct((_MG, 2, H), _F)),
    )(y2, m1)


def _mlp_b_body(t1_ref, a_ref, b_ref, m2_ref, t2_ref, st_ref):
    h = jnp.maximum(t1_ref[...] * a_ref[...] + b_ref[...], 0.0)
    t = _dot(h, m2_ref[...])
    t2_ref[...] = t
    st_ref[0] = _stats(t)


def _mlp_b(t1, a, b, m2):
    return pl.pallas_call(
        _mlp_b_body,
        grid=(_MG,),
        in_specs=[
            pl.BlockSpec((_BR, H), lambda i: (i, 0)),
            pl.BlockSpec((H,), lambda i: (0,)),
            pl.BlockSpec((H,), lambda i: (0,)),
            pl.BlockSpec((H, H), lambda i: (0, 0)),
        ],
        out_specs=(pl.BlockSpec((_BR, H), lambda i: (i, 0)),
                   pl.BlockSpec((1, 2, H), lambda i: (i, 0, 0))),
        out_shape=(jax.ShapeDtypeStruct((N, H), _F),
                   jax.ShapeDtypeStruct((_MG, 2, H), _F)),
    )(t1, a, b, m2)


def _mlp_c_body(t2_ref, a_ref, b_ref, m3_ref, bias_ref, o_ref):
    h = jnp.maximum(t2_ref[...] * a_ref[...] + b_ref[...], 0.0)
    o_ref[...] = _dot(h, m3_ref[...]) + bias_ref[...]


def _mlp_c(t2, a, b, m3, bias3):
    return pl.pallas_call(
        _mlp_c_body,
        grid=(_MG,),
        in_specs=[
            pl.BlockSpec((_BR, H), lambda i: (i, 0)),
            pl.BlockSpec((H,), lambda i: (0,)),
            pl.BlockSpec((H,), lambda i: (0,)),
            pl.BlockSpec((H, D), lambda i: (0, 0)),
            pl.BlockSpec((D,), lambda i: (0,)),
        ],
        out_specs=pl.BlockSpec((_BR, D), lambda i: (i, 0)),
        out_shape=jax.ShapeDtypeStruct((N, D), _F),
    )(t2, a, b, m3, bias3)


def kernel(node_rep, edge_rep, edge_attr, edge_index, W1, W2, W3, We,
           bn_g, bn_b, M1, g1, b1, M2, g2, b2, M3, bias3):
    src = edge_index[0].astype(jnp.int32)
    dst = edge_index[1].astype(jnp.int32)

    xw1, xw2 = _node_mm(node_rep, W1, W2)
    edense = _edge_mm(edge_rep, edge_attr.T, W3, We)

    msgs, acc = _sc_pass1(xw1, xw2, edense, src, dst)

    scale, shift = _bn_ab(acc, bn_g, bn_b, E)
    ab = jnp.stack([scale, shift])

    y2 = _sc_pass2(msgs, dst, ab)

    t1, st1 = _mlp_a(y2, M1)
    a1, s1 = _bn_ab(st1, g1, b1, N)
    t2, st2 = _mlp_b(t1, a1, s1, M2)
    a2, s2 = _bn_ab(st2, g2, b2, N)
    return _mlp_c(t2, a2, s2, M3, bias3)


def _bn_ab(st, g, b, n):
    ssum = jnp.sum(st[:, 0, :], axis=0)
    ssq = jnp.sum(st[:, 1, :], axis=0)
    mean = ssum / n
    var = ssq / n - mean * mean
    scale = g * lax.rsqrt(var + EPS)
    return scale, b - mean * scale


# speedup vs baseline: 4.0405x; 1.0980x over previous
import functools

import jax
import jax.numpy as jnp
from jax import lax
from jax.experimental import pallas as pl
from jax.experimental.pallas import tpu as pltpu
from jax.experimental.pallas import tpu_sc as plsc

N = 10000
E = 320000
D = 128
DE = 16
H = 256
EPS = 1e-5

NC = 2
NS = 16
NW = NC * NS
EPW = E // NW
BLK = 80
NBLK = EPW // BLK
BLK2 = 80
NBLK2 = EPW // BLK2
RPS = 624

_HI = lax.Precision.HIGHEST
_F = jnp.float32


def _dot(a, b):
    return jnp.dot(a, b, preferred_element_type=_F, precision=_HI)


_BN_ROWS = 2000

def _node_mm_body(x_ref, w1_ref, w2_ref, o1_ref, o2_ref):
    x = x_ref[...]
    o1_ref[...] = _dot(x, w1_ref[...])
    o2_ref[...] = _dot(x, w2_ref[...])


def _node_mm(x, w1, w2):
    return pl.pallas_call(
        _node_mm_body,
        grid=(N // _BN_ROWS,),
        in_specs=[
            pl.BlockSpec((_BN_ROWS, D), lambda i: (i, 0)),
            pl.BlockSpec((D, D), lambda i: (0, 0)),
            pl.BlockSpec((D, D), lambda i: (0, 0)),
        ],
        out_specs=(pl.BlockSpec((_BN_ROWS, D), lambda i: (i, 0)),
                   pl.BlockSpec((_BN_ROWS, D), lambda i: (i, 0))),
        out_shape=(jax.ShapeDtypeStruct((N, D), _F),
                   jax.ShapeDtypeStruct((N, D), _F)),
    )(x, w1, w2)


_BE = 3200

def _edge_mm_body(er_ref, eat_ref, w3_ref, we_ref, o_ref):
    ea_part = lax.dot_general(eat_ref[...], we_ref[...],
                              (((0,), (0,)), ((), ())),
                              preferred_element_type=_F,
                              precision=lax.Precision.DEFAULT)
    ed = jnp.dot(er_ref[...], w3_ref[...], preferred_element_type=_F,
                 precision=lax.Precision.DEFAULT) + ea_part
    edp = ed.reshape(_BE // 2, 2, D)
    lo = lax.bitcast_convert_type(edp[:, 0, :].astype(jnp.bfloat16), jnp.uint16)
    hi = lax.bitcast_convert_type(edp[:, 1, :].astype(jnp.bfloat16), jnp.uint16)
    o_ref[...] = lo.astype(jnp.int32) | (hi.astype(jnp.int32) << 16)


def _edge_mm(edge_rep, edge_attr_t, w3, we):
    return pl.pallas_call(
        _edge_mm_body,
        grid=(E // _BE,),
        in_specs=[
            pl.BlockSpec((_BE, D), lambda i: (i, 0)),
            pl.BlockSpec((DE, _BE), lambda i: (0, i)),
            pl.BlockSpec((D, D), lambda i: (0, 0)),
            pl.BlockSpec((DE, D), lambda i: (0, 0)),
        ],
        out_specs=pl.BlockSpec((_BE // 2, D), lambda i: (i, 0)),
        out_shape=jax.ShapeDtypeStruct((E // 2, D), jnp.int32),
    )(edge_rep, edge_attr_t, w3, we)


_mesh = plsc.VectorSubcoreMesh(core_axis_name="c", subcore_axis_name="s")


def _bf16_unpack2(w):
    f0 = lax.bitcast_convert_type(w << 16, _F)
    f1 = lax.bitcast_convert_type(w & jnp.int32(-65536), _F)
    return f0, f1


@functools.partial(
    pl.kernel,
    mesh=_mesh,
    out_type=(jax.ShapeDtypeStruct((E, D), _F),
              jax.ShapeDtypeStruct((NW, 2, D), _F)),
    scratch_types=[
        (pltpu.VMEM((BLK,), jnp.int32),) * 2,
        (pltpu.VMEM((BLK,), jnp.int32),) * 2,
        (pltpu.VMEM((BLK, D), _F),) * 2,
        (pltpu.VMEM((BLK, D), _F),) * 2,
        (pltpu.VMEM((BLK // 2, D), jnp.int32),) * 2,
        pltpu.VMEM((2, D), _F),
        (pltpu.SemaphoreType.DMA,) * 2,
        (pltpu.SemaphoreType.DMA,) * 2,
        (pltpu.SemaphoreType.DMA,) * 2,
    ],
)
def _sc_pass1(xw1, xw2, edense, src_hbm, dst_hbm, msg_out, acc_out,
              src_v, dst_v, g1_v, g2_v, ed_v, st_v, sem_i, sem_g, sem_o):
    c = lax.axis_index("c")
    s = lax.axis_index("s")
    wid = s * NC + c
    base = wid * EPW

    def idx_copies(j, b):
        off = base + j * BLK
        return (pltpu.make_async_copy(src_hbm.at[pl.ds(off, BLK)], src_v[b], sem_i[b]),
                pltpu.make_async_copy(dst_hbm.at[pl.ds(off, BLK)], dst_v[b], sem_i[b]))

    def gat_copies(j, b):
        offp = wid * (EPW // 2) + j * (BLK // 2)
        return (pltpu.make_async_copy(xw1.at[src_v[b]], g1_v[b], sem_g[b]),
                pltpu.make_async_copy(xw2.at[dst_v[b]], g2_v[b], sem_g[b]),
                pltpu.make_async_copy(edense.at[pl.ds(offp, BLK // 2)], ed_v[b], sem_g[b]))

    def out_copy(j, b):
        off = base + j * BLK
        return pltpu.make_async_copy(g1_v[b], msg_out.at[pl.ds(off, BLK)], sem_o[b])

    for cp in idx_copies(0, 0) + idx_copies(1, 1):
        cp.start()
    for cp in idx_copies(0, 0):
        cp.wait()
    for cp in gat_copies(0, 0):
        cp.start()

    def step(j, b, carry):
        q = 1 - b
        for cp in gat_copies(j, b):
            cp.wait()

        @pl.when(j + 1 < NBLK)
        def _launch_next():
            for cp in idx_copies(j + 1, q):
                cp.wait()

            @pl.when(j >= 1)
            def _drain_prev_out():
                out_copy(j - 1, q).wait()

            for cp in gat_copies(j + 1, q):
                cp.start()

        @pl.when(j + 2 < NBLK)
        def _prefetch_idx():
            for cp in idx_copies(j + 2, b):
                cp.start()

        def rowpair(t, acc):
            r0 = 2 * t
            r1 = r0 + 1
            new = list(acc)
            for f in range(8):
                sl = pl.ds(f * 16, 16)
                e0, e1 = _bf16_unpack2(ed_v[b][t, sl])
                v0 = g1_v[b][r0, sl] + g2_v[b][r0, sl] + e0
                v1 = g1_v[b][r1, sl] + g2_v[b][r1, sl] + e1
                g1_v[b][r0, sl] = v0
                g1_v[b][r1, sl] = v1
                new[f] = acc[f] + (v0 + v1)
                new[8 + f] = acc[8 + f] + (v0 * v0 + v1 * v1)
            return tuple(new)

        carry = lax.fori_loop(0, BLK // 2, rowpair, carry)
        out_copy(j, b).start()
        return carry

    def pair(io, carry):
        carry = step(2 * io, 0, carry)
        return step(2 * io + 1, 1, carry)

    zero = jnp.zeros((16,), _F)
    acc = lax.fori_loop(0, NBLK // 2, pair, tuple(zero for _ in range(16)))
    acc = step(NBLK - 1, 0, acc)
    out_copy(NBLK - 2, 1).wait()
    out_copy(NBLK - 1, 0).wait()
    for f in range(8):
        st_v[0, pl.ds(f * 16, 16)] = acc[f]
        st_v[1, pl.ds(f * 16, 16)] = acc[8 + f]
    pltpu.sync_copy(st_v, acc_out.at[wid])


@functools.partial(
    pl.kernel,
    mesh=_mesh,
    out_type=jax.ShapeDtypeStruct((NC, N, D), _F),
    scratch_types=[
        (pltpu.VMEM((BLK2,), jnp.int32),) * 3,
        (pltpu.VMEM((BLK2, D), _F),) * 3,
        pltpu.VMEM((2, D), _F),
        pltpu.VMEM((104, D), _F),
        pltpu.VMEM_SHARED((N, D), _F),
        (pltpu.SemaphoreType.DMA,) * 3,
        (pltpu.SemaphoreType.DMA,) * 3,
    ],
)
def _sc_pass2(msg_hbm, dst_hbm, ab_hbm, y_out,
              idx_v, m_v, ab_v, z_v, ysh, sem_i, sem_s):
    c = lax.axis_index("c")
    s = lax.axis_index("s")
    wid = s * NC + c
    base = wid * EPW

    pltpu.sync_copy(ab_hbm, ab_v)
    a = [ab_v[0, pl.ds(f * 16, 16)] for f in range(8)]
    b = [ab_v[1, pl.ds(f * 16, 16)] for f in range(8)]

    zero = jnp.zeros((16,), _F)

    def zrow(r, _):
        for f in range(8):
            z_v[r, pl.ds(f * 16, 16)] = zero
        return 0

    lax.fori_loop(0, 104, zrow, 0)

    def zchunk(k, _):
        pltpu.sync_copy(z_v, ysh.at[pl.ds(s * RPS + k * 104, 104)])
        return 0

    lax.fori_loop(0, RPS // 104, zchunk, 0)

    @pl.when(s == NS - 1)
    def _zero_tail():
        pltpu.sync_copy(z_v.at[pl.ds(0, 16)], ysh.at[pl.ds(NS * RPS, 16)])

    plsc.subcore_barrier()

    def in_copies(j, p):
        off = base + j * BLK2
        return (pltpu.make_async_copy(dst_hbm.at[pl.ds(off, BLK2)], idx_v[p], sem_i[p]),
                pltpu.make_async_copy(msg_hbm.at[pl.ds(off, BLK2)], m_v[p], sem_i[p]))

    def sc_copy(p):
        return pltpu.async_copy(m_v[p], ysh.at[idx_v[p]], sem_s[p], add=True)

    def sc_wait(p):
        pltpu.make_async_copy(m_v[p], ysh.at[idx_v[p]], sem_s[p]).wait()

    for cp in in_copies(0, 0) + in_copies(1, 1):
        cp.start()

    def step(j, p):
        for cp in in_copies(j, p):
            cp.wait()

        def row(r, _2):
            for f in range(8):
                sl = pl.ds(f * 16, 16)
                m_v[p][r, sl] = jnp.maximum(m_v[p][r, sl] * a[f] + b[f], 0.0)
            return 0

        lax.fori_loop(0, BLK2, row, 0)

        @pl.when(j >= 1)
        def _drain_prev_scatter():
            sc_wait((p + 2) % 3)

        @pl.when(j + 2 < NBLK2)
        def _prefetch_in():
            for cp in in_copies(j + 2, (p + 2) % 3):
                cp.start()

        sc_copy(p)

    def triple(io, _):
        step(3 * io, 0)
        step(3 * io + 1, 1)
        step(3 * io + 2, 2)
        return 0

    lax.fori_loop(0, NBLK2 // 3, triple, 0)
    step(NBLK2 - 2, 0)
    step(NBLK2 - 1, 1)
    sc_wait(1)
    plsc.subcore_barrier()
    pltpu.sync_copy(ysh.at[pl.ds(s * RPS, RPS)],
                    y_out.at[c, pl.ds(s * RPS, RPS)])

    @pl.when(s == NS - 1)
    def _out_tail():
        pltpu.sync_copy(ysh.at[pl.ds(NS * RPS, 16)],
                        y_out.at[c, pl.ds(NS * RPS, 16)])


_BR = 2000
_MG = N // _BR


def _stats(t):
    s0 = jnp.sum(t, axis=0, keepdims=True)
    s1 = jnp.sum(t * t, axis=0, keepdims=True)
    return jnp.concatenate([s0, s1], axis=0)


def _mlp_a_body(y2_ref, m1_ref, t1_ref, st_ref):
    y = y2_ref[0] + y2_ref[1]
    t = _dot(y, m1_ref[...])
    t1_ref[...] = t
    st_ref[0] = _stats(t)


def _mlp_a(y2, m1):
    return pl.pallas_call(
        _mlp_a_body,
        grid=(_MG,),
        in_specs=[
            pl.BlockSpec((NC, _BR, D), lambda i: (0, i, 0)),
            pl.BlockSpec((D, H), lambda i: (0, 0)),
        ],
        out_specs=(pl.BlockSpec((_BR, H), lambda i: (i, 0)),
                   pl.BlockSpec((1, 2, H), lambda i: (i, 0, 0))),
        out_shape=(jax.ShapeDtypeStruct((N, H), _F),
                   jax.ShapeDtypeStruct((_MG, 2, H), _F)),
    )(y2, m1)


def _mlp_b_body(t1_ref, a_ref, b_ref, m2_ref, t2_ref, st_ref):
    h = jnp.maximum(t1_ref[...] * a_ref[...] + b_ref[...], 0.0)
    t = _dot(h, m2_ref[...])
    t2_ref[...] = t
    st_ref[0] = _stats(t)


def _mlp_b(t1, a, b, m2):
    return pl.pallas_call(
        _mlp_b_body,
        grid=(_MG,),
        in_specs=[
            pl.BlockSpec((_BR, H), lambda i: (i, 0)),
            pl.BlockSpec((H,), lambda i: (0,)),
            pl.BlockSpec((H,), lambda i: (0,)),
            pl.BlockSpec((H, H), lambda i: (0, 0)),
        ],
        out_specs=(pl.BlockSpec((_BR, H), lambda i: (i, 0)),
                   pl.BlockSpec((1, 2, H), lambda i: (i, 0, 0))),
        out_shape=(jax.ShapeDtypeStruct((N, H), _F),
                   jax.ShapeDtypeStruct((_MG, 2, H), _F)),
    )(t1, a, b, m2)


def _mlp_c_body(t2_ref, a_ref, b_ref, m3_ref, bias_ref, o_ref):
    h = jnp.maximum(t2_ref[...] * a_ref[...] + b_ref[...], 0.0)
    o_ref[...] = _dot(h, m3_ref[...]) + bias_ref[...]


def _mlp_c(t2, a, b, m3, bias3):
    return pl.pallas_call(
        _mlp_c_body,
        grid=(_MG,),
        in_specs=[
            pl.BlockSpec((_BR, H), lambda i: (i, 0)),
            pl.BlockSpec((H,), lambda i: (0,)),
            pl.BlockSpec((H,), lambda i: (0,)),
            pl.BlockSpec((H, D), lambda i: (0, 0)),
            pl.BlockSpec((D,), lambda i: (0,)),
        ],
        out_specs=pl.BlockSpec((_BR, D), lambda i: (i, 0)),
        out_shape=jax.ShapeDtypeStruct((N, D), _F),
    )(t2, a, b, m3, bias3)


def _bn_ab(st, g, b, n):
    ssum = jnp.sum(st[:, 0, :], axis=0)
    ssq = jnp.sum(st[:, 1, :], axis=0)
    mean = ssum / n
    var = ssq / n - mean * mean
    scale = g * lax.rsqrt(var + EPS)
    return scale, b - mean * scale


def kernel(node_rep, edge_rep, edge_attr, edge_index, W1, W2, W3, We,
           bn_g, bn_b, M1, g1, b1, M2, g2, b2, M3, bias3):
    src = edge_index[0].astype(jnp.int32)
    dst = edge_index[1].astype(jnp.int32)

    xw1, xw2 = _node_mm(node_rep, W1, W2)
    edense = _edge_mm(edge_rep, edge_attr.T, W3, We)

    msgs, acc = _sc_pass1(xw1, xw2, edense, src, dst)

    scale, shift = _bn_ab(acc, bn_g, bn_b, E)
    ab = jnp.stack([scale, shift])

    y2 = _sc_pass2(msgs, dst, ab)

    t1, st1 = _mlp_a(y2, M1)
    a1, s1 = _bn_ab(st1, g1, b1, N)
    t2, st2 = _mlp_b(t1, a1, s1, M2)
    a2, s2 = _bn_ab(st2, g2, b2, N)
    return _mlp_c(t2, a2, s2, M3, bias3)


# speedup vs baseline: 4.3393x; 1.0740x over previous
import functools

import jax
import jax.numpy as jnp
from jax import lax
from jax.experimental import pallas as pl
from jax.experimental.pallas import tpu as pltpu
from jax.experimental.pallas import tpu_sc as plsc

N = 10000
E = 320000
D = 128
DE = 16
H = 256
EPS = 1e-5

NC = 2
NS = 16
NW = NC * NS
EPW = E // NW
BLK = 80
NBLK = EPW // BLK
BLK2 = 80
NBLK2 = EPW // BLK2
RPS = 624

_HI = lax.Precision.HIGHEST
_F = jnp.float32


def _dot(a, b):
    return jnp.dot(a, b, preferred_element_type=_F, precision=_HI)


_BN_ROWS = 2000

def _node_mm_body(x_ref, w1_ref, w2_ref, o1_ref, o2_ref):
    x = x_ref[...]
    o1_ref[...] = _dot(x, w1_ref[...])
    o2_ref[...] = _dot(x, w2_ref[...])


def _node_mm(x, w1, w2):
    return pl.pallas_call(
        _node_mm_body,
        grid=(N // _BN_ROWS,),
        in_specs=[
            pl.BlockSpec((_BN_ROWS, D), lambda i: (i, 0)),
            pl.BlockSpec((D, D), lambda i: (0, 0)),
            pl.BlockSpec((D, D), lambda i: (0, 0)),
        ],
        out_specs=(pl.BlockSpec((_BN_ROWS, D), lambda i: (i, 0)),
                   pl.BlockSpec((_BN_ROWS, D), lambda i: (i, 0))),
        out_shape=(jax.ShapeDtypeStruct((N, D), _F),
                   jax.ShapeDtypeStruct((N, D), _F)),
    )(x, w1, w2)


_BE = 6400

def _edge_mm_body(er_ref, eat_ref, w3_ref, we_ref, o_ref):
    ea_part = lax.dot_general(eat_ref[...], we_ref[...],
                              (((0,), (0,)), ((), ())),
                              preferred_element_type=_F,
                              precision=lax.Precision.DEFAULT)
    ed = jnp.dot(er_ref[...], w3_ref[...], preferred_element_type=_F,
                 precision=lax.Precision.DEFAULT) + ea_part
    edp = ed.reshape(_BE // 2, 2, D)
    lo = lax.bitcast_convert_type(edp[:, 0, :].astype(jnp.bfloat16), jnp.uint16)
    hi = lax.bitcast_convert_type(edp[:, 1, :].astype(jnp.bfloat16), jnp.uint16)
    o_ref[...] = lo.astype(jnp.int32) | (hi.astype(jnp.int32) << 16)


def _edge_mm(edge_rep, edge_attr_t, w3, we):
    return pl.pallas_call(
        _edge_mm_body,
        grid=(E // _BE,),
        in_specs=[
            pl.BlockSpec((_BE, D), lambda i: (i, 0)),
            pl.BlockSpec((DE, _BE), lambda i: (0, i)),
            pl.BlockSpec((D, D), lambda i: (0, 0)),
            pl.BlockSpec((DE, D), lambda i: (0, 0)),
        ],
        out_specs=pl.BlockSpec((_BE // 2, D), lambda i: (i, 0)),
        out_shape=jax.ShapeDtypeStruct((E // 2, D), jnp.int32),
    )(edge_rep, edge_attr_t, w3, we)


_mesh = plsc.VectorSubcoreMesh(core_axis_name="c", subcore_axis_name="s")


def _bf16_unpack2(w):
    f0 = lax.bitcast_convert_type(w << 16, _F)
    f1 = lax.bitcast_convert_type(w & jnp.int32(-65536), _F)
    return f0, f1


@functools.partial(
    pl.kernel,
    mesh=_mesh,
    out_type=(jax.ShapeDtypeStruct((E, D), _F),
              jax.ShapeDtypeStruct((NW, 2, D), _F)),
    scratch_types=[
        (pltpu.VMEM((BLK,), jnp.int32),) * 2,
        (pltpu.VMEM((BLK,), jnp.int32),) * 2,
        (pltpu.VMEM((BLK, D), _F),) * 2,
        (pltpu.VMEM((BLK, D), _F),) * 2,
        (pltpu.VMEM((BLK // 2, D), jnp.int32),) * 2,
        pltpu.VMEM((2, D), _F),
        (pltpu.SemaphoreType.DMA,) * 2,
        (pltpu.SemaphoreType.DMA,) * 2,
        (pltpu.SemaphoreType.DMA,) * 2,
    ],
)
def _sc_pass1(xw1, xw2, edense, src_hbm, dst_hbm, msg_out, acc_out,
              src_v, dst_v, g1_v, g2_v, ed_v, st_v, sem_i, sem_g, sem_o):
    c = lax.axis_index("c")
    s = lax.axis_index("s")
    wid = s * NC + c
    base = wid * EPW

    def idx_copies(j, b):
        off = base + j * BLK
        return (pltpu.make_async_copy(src_hbm.at[pl.ds(off, BLK)], src_v[b], sem_i[b]),
                pltpu.make_async_copy(dst_hbm.at[pl.ds(off, BLK)], dst_v[b], sem_i[b]))

    def gat_copies(j, b):
        offp = wid * (EPW // 2) + j * (BLK // 2)
        return (pltpu.make_async_copy(xw1.at[src_v[b]], g1_v[b], sem_g[b]),
                pltpu.make_async_copy(xw2.at[dst_v[b]], g2_v[b], sem_g[b]),
                pltpu.make_async_copy(edense.at[pl.ds(offp, BLK // 2)], ed_v[b], sem_g[b]))

    def out_copy(j, b):
        off = base + j * BLK
        return pltpu.make_async_copy(g1_v[b], msg_out.at[pl.ds(off, BLK)], sem_o[b])

    for cp in idx_copies(0, 0) + idx_copies(1, 1):
        cp.start()
    for cp in idx_copies(0, 0):
        cp.wait()
    for cp in gat_copies(0, 0):
        cp.start()

    def step(j, b, carry):
        q = 1 - b
        for cp in gat_copies(j, b):
            cp.wait()

        @pl.when(j + 1 < NBLK)
        def _launch_next():
            for cp in idx_copies(j + 1, q):
                cp.wait()

            @pl.when(j >= 1)
            def _drain_prev_out():
                out_copy(j - 1, q).wait()

            for cp in gat_copies(j + 1, q):
                cp.start()

        @pl.when(j + 2 < NBLK)
        def _prefetch_idx():
            for cp in idx_copies(j + 2, b):
                cp.start()

        def rowpair(t, acc):
            r0 = 2 * t
            r1 = r0 + 1
            new = list(acc)
            for f in range(8):
                sl = pl.ds(f * 16, 16)
                e0, e1 = _bf16_unpack2(ed_v[b][t, sl])
                v0 = g1_v[b][r0, sl] + g2_v[b][r0, sl] + e0
                v1 = g1_v[b][r1, sl] + g2_v[b][r1, sl] + e1
                g1_v[b][r0, sl] = v0
                g1_v[b][r1, sl] = v1
                new[f] = acc[f] + (v0 + v1)
                new[8 + f] = acc[8 + f] + (v0 * v0 + v1 * v1)
            return tuple(new)

        carry = lax.fori_loop(0, BLK // 2, rowpair, carry)
        out_copy(j, b).start()
        return carry

    def pair(io, carry):
        carry = step(2 * io, 0, carry)
        return step(2 * io + 1, 1, carry)

    zero = jnp.zeros((16,), _F)
    acc = lax.fori_loop(0, NBLK // 2, pair, tuple(zero for _ in range(16)))
    acc = step(NBLK - 1, 0, acc)
    out_copy(NBLK - 2, 1).wait()
    out_copy(NBLK - 1, 0).wait()
    for f in range(8):
        st_v[0, pl.ds(f * 16, 16)] = acc[f]
        st_v[1, pl.ds(f * 16, 16)] = acc[8 + f]
    pltpu.sync_copy(st_v, acc_out.at[wid])


@functools.partial(
    pl.kernel,
    mesh=_mesh,
    out_type=jax.ShapeDtypeStruct((NC, N, D), _F),
    scratch_types=[
        (pltpu.VMEM((BLK2,), jnp.int32),) * 3,
        (pltpu.VMEM((BLK2, D), _F),) * 3,
        pltpu.VMEM((2, D), _F),
        pltpu.VMEM((104, D), _F),
        pltpu.VMEM_SHARED((N, D), _F),
        (pltpu.SemaphoreType.DMA,) * 3,
        (pltpu.SemaphoreType.DMA,) * 3,
    ],
)
def _sc_pass2(msg_hbm, dst_hbm, ab_hbm, y_out,
              idx_v, m_v, ab_v, z_v, ysh, sem_i, sem_s):
    c = lax.axis_index("c")
    s = lax.axis_index("s")
    wid = s * NC + c
    base = wid * EPW

    pltpu.sync_copy(ab_hbm, ab_v)
    a = [ab_v[0, pl.ds(f * 16, 16)] for f in range(8)]
    b = [ab_v[1, pl.ds(f * 16, 16)] for f in range(8)]

    zero = jnp.zeros((16,), _F)

    def zrow(r, _):
        for f in range(8):
            z_v[r, pl.ds(f * 16, 16)] = zero
        return 0

    lax.fori_loop(0, 104, zrow, 0)

    def zchunk(k, _):
        pltpu.sync_copy(z_v, ysh.at[pl.ds(s * RPS + k * 104, 104)])
        return 0

    lax.fori_loop(0, RPS // 104, zchunk, 0)

    @pl.when(s == NS - 1)
    def _zero_tail():
        pltpu.sync_copy(z_v.at[pl.ds(0, 16)], ysh.at[pl.ds(NS * RPS, 16)])

    plsc.subcore_barrier()

    def in_copies(j, p):
        off = base + j * BLK2
        return (pltpu.make_async_copy(dst_hbm.at[pl.ds(off, BLK2)], idx_v[p], sem_i[p]),
                pltpu.make_async_copy(msg_hbm.at[pl.ds(off, BLK2)], m_v[p], sem_i[p]))

    def sc_copy(p):
        return pltpu.async_copy(m_v[p], ysh.at[idx_v[p]], sem_s[p], add=True)

    def sc_wait(p):
        pltpu.make_async_copy(m_v[p], ysh.at[idx_v[p]], sem_s[p]).wait()

    for cp in in_copies(0, 0) + in_copies(1, 1):
        cp.start()

    def step(j, p):
        for cp in in_copies(j, p):
            cp.wait()

        def row(r, _2):
            for f in range(8):
                sl = pl.ds(f * 16, 16)
                m_v[p][r, sl] = jnp.maximum(m_v[p][r, sl] * a[f] + b[f], 0.0)
            return 0

        lax.fori_loop(0, BLK2, row, 0)

        @pl.when(j >= 1)
        def _drain_prev_scatter():
            sc_wait((p + 2) % 3)

        @pl.when(j + 2 < NBLK2)
        def _prefetch_in():
            for cp in in_copies(j + 2, (p + 2) % 3):
                cp.start()

        sc_copy(p)

    def triple(io, _):
        step(3 * io, 0)
        step(3 * io + 1, 1)
        step(3 * io + 2, 2)
        return 0

    lax.fori_loop(0, NBLK2 // 3, triple, 0)
    step(NBLK2 - 2, 0)
    step(NBLK2 - 1, 1)
    sc_wait(1)
    plsc.subcore_barrier()
    pltpu.sync_copy(ysh.at[pl.ds(s * RPS, RPS)],
                    y_out.at[c, pl.ds(s * RPS, RPS)])

    @pl.when(s == NS - 1)
    def _out_tail():
        pltpu.sync_copy(ysh.at[pl.ds(NS * RPS, 16)],
                        y_out.at[c, pl.ds(NS * RPS, 16)])


_BR = 2000
_MG = N // _BR


def _stats(t):
    s0 = jnp.sum(t, axis=0, keepdims=True)
    s1 = jnp.sum(t * t, axis=0, keepdims=True)
    return jnp.concatenate([s0, s1], axis=0)


def _mlp_a_body(y2_ref, m1_ref, t1_ref, st_ref):
    y = y2_ref[0] + y2_ref[1]
    t = _dot(y, m1_ref[...])
    t1_ref[...] = t
    st_ref[0] = _stats(t)


def _mlp_a(y2, m1):
    return pl.pallas_call(
        _mlp_a_body,
        grid=(_MG,),
        in_specs=[
            pl.BlockSpec((NC, _BR, D), lambda i: (0, i, 0)),
            pl.BlockSpec((D, H), lambda i: (0, 0)),
        ],
        out_specs=(pl.BlockSpec((_BR, H), lambda i: (i, 0)),
                   pl.BlockSpec((1, 2, H), lambda i: (i, 0, 0))),
        out_shape=(jax.ShapeDtypeStruct((N, H), _F),
                   jax.ShapeDtypeStruct((_MG, 2, H), _F)),
    )(y2, m1)


def _mlp_b_body(t1_ref, a_ref, b_ref, m2_ref, t2_ref, st_ref):
    h = jnp.maximum(t1_ref[...] * a_ref[...] + b_ref[...], 0.0)
    t = _dot(h, m2_ref[...])
    t2_ref[...] = t
    st_ref[0] = _stats(t)


def _mlp_b(t1, a, b, m2):
    return pl.pallas_call(
        _mlp_b_body,
        grid=(_MG,),
        in_specs=[
            pl.BlockSpec((_BR, H), lambda i: (i, 0)),
            pl.BlockSpec((H,), lambda i: (0,)),
            pl.BlockSpec((H,), lambda i: (0,)),
            pl.BlockSpec((H, H), lambda i: (0, 0)),
        ],
        out_specs=(pl.BlockSpec((_BR, H), lambda i: (i, 0)),
                   pl.BlockSpec((1, 2, H), lambda i: (i, 0, 0))),
        out_shape=(jax.ShapeDtypeStruct((N, H), _F),
                   jax.ShapeDtypeStruct((_MG, 2, H), _F)),
    )(t1, a, b, m2)


def _mlp_c_body(t2_ref, a_ref, b_ref, m3_ref, bias_ref, o_ref):
    h = jnp.maximum(t2_ref[...] * a_ref[...] + b_ref[...], 0.0)
    o_ref[...] = _dot(h, m3_ref[...]) + bias_ref[...]


def _mlp_c(t2, a, b, m3, bias3):
    return pl.pallas_call(
        _mlp_c_body,
        grid=(_MG,),
        in_specs=[
            pl.BlockSpec((_BR, H), lambda i: (i, 0)),
            pl.BlockSpec((H,), lambda i: (0,)),
            pl.BlockSpec((H,), lambda i: (0,)),
            pl.BlockSpec((H, D), lambda i: (0, 0)),
            pl.BlockSpec((D,), lambda i: (0,)),
        ],
        out_specs=pl.BlockSpec((_BR, D), lambda i: (i, 0)),
        out_shape=jax.ShapeDtypeStruct((N, D), _F),
    )(t2, a, b, m3, bias3)


def _bn_ab(st, g, b, n):
    ssum = jnp.sum(st[:, 0, :], axis=0)
    ssq = jnp.sum(st[:, 1, :], axis=0)
    mean = ssum / n
    var = ssq / n - mean * mean
    scale = g * lax.rsqrt(var + EPS)
    return scale, b - mean * scale


def kernel(node_rep, edge_rep, edge_attr, edge_index, W1, W2, W3, We,
           bn_g, bn_b, M1, g1, b1, M2, g2, b2, M3, bias3):
    src = edge_index[0].astype(jnp.int32)
    dst = edge_index[1].astype(jnp.int32)

    xw1, xw2 = _node_mm(node_rep, W1, W2)
    edense = _edge_mm(edge_rep, edge_attr.T, W3, We)

    msgs, acc = _sc_pass1(xw1, xw2, edense, src, dst)

    scale, shift = _bn_ab(acc, bn_g, bn_b, E)
    ab = jnp.stack([scale, shift])

    y2 = _sc_pass2(msgs, dst, ab)

    t1, st1 = _mlp_a(y2, M1)
    a1, s1 = _bn_ab(st1, g1, b1, N)
    t2, st2 = _mlp_b(t1, a1, s1, M2)
    a2, s2 = _bn_ab(st2, g2, b2, N)
    return _mlp_c(t2, a2, s2, M3, bias3)
